# Initial kernel scaffold; baseline (speedup 1.0000x reference)
import jax
import jax.numpy as jnp
from jax.experimental import pallas as pl


def kernel(node_features, frac_coords, lattices, edge_index, edge2graph, frac_diff, l_f_features, We1, be1, We2, be2, Wn1, bn1, Wn2, bn2):
    raise NotImplementedError("write your pallas kernel here")


# speedup vs baseline: 2.6032x; 2.6032x over previous
import functools

import jax
import jax.numpy as jnp
from jax import lax
from jax.experimental import pallas as pl
from jax.experimental.pallas import tpu as pltpu
from jax.experimental.pallas import tpu_sc as plsc

N = 10000
H = 128
HH = H // 2
G = 128
E = 320000

NC = 2
NS = 16
NW = NC * NS
CHUNK = 128
KPT = 80
E_PAD = NW * KPT * CHUNK
NP = 10240
NPB = NP // CHUNK
LANES = 16


def _proj_body(nf_ref, w1a_ref, w1b_ref, lat_ref, w1c_ref, be1_ref,
               p_ref, q_ref, lp_ref):
    nf = nf_ref[...]
    p_ref[...] = jnp.dot(nf, w1a_ref[...], preferred_element_type=jnp.float32)
    q_ref[...] = jnp.dot(nf, w1b_ref[...], preferred_element_type=jnp.float32)
    lp_ref[...] = (jnp.dot(lat_ref[...], w1c_ref[...],
                           preferred_element_type=jnp.float32) + be1_ref[...])


_BN1 = 2048


def _stage1(nf_pad, w1a, w1b, lattices, w1c, be1r):
    return pl.pallas_call(
        _proj_body,
        grid=(NP // _BN1,),
        in_specs=[
            pl.BlockSpec((_BN1, H), lambda i: (i, 0)),
            pl.BlockSpec((H, H), lambda i: (0, 0)),
            pl.BlockSpec((H, H), lambda i: (0, 0)),
            pl.BlockSpec((G, 6), lambda i: (0, 0)),
            pl.BlockSpec((6, H), lambda i: (0, 0)),
            pl.BlockSpec((1, H), lambda i: (0, 0)),
        ],
        out_specs=[
            pl.BlockSpec((_BN1, H), lambda i: (i, 0)),
            pl.BlockSpec((_BN1, H), lambda i: (i, 0)),
            pl.BlockSpec((G, H), lambda i: (0, 0)),
        ],
        out_shape=[
            jax.ShapeDtypeStruct((NP, H), jnp.float32),
            jax.ShapeDtypeStruct((NP, H), jnp.float32),
            jax.ShapeDtypeStruct((G, H), jnp.float32),
        ],
    )(nf_pad, w1a, w1b, lattices, w1c, be1r)


def _gather_body(p_hbm, q_hbm, src_hbm, dst_hbm, pre_hbm,
                 sidx, didx, bufp, bufq, semp, semq):
    c = lax.axis_index("c")
    s = lax.axis_index("s")
    wid = s * NC + c
    base = wid * KPT
    pltpu.sync_copy(src_hbm.at[pl.ds(base, KPT)], sidx)
    pltpu.sync_copy(dst_hbm.at[pl.ds(base, KPT)], didx)

    def chunk_body(j, carry):
        cp = pltpu.async_copy(p_hbm.at[sidx.at[j]], bufp, semp)
        cq = pltpu.async_copy(q_hbm.at[didx.at[j]], bufq, semq)
        cp.wait()
        cq.wait()

        def row_body(r, carry2):
            for cc in range(H // LANES):
                sl = pl.ds(cc * LANES, LANES)
                bufp[r, sl] = bufp[r, sl] + bufq[r, sl]
            return carry2

        lax.fori_loop(0, CHUNK, row_body, 0)
        pltpu.sync_copy(bufp, pre_hbm.at[pl.ds((base + j) * CHUNK, CHUNK)])
        return carry

    lax.fori_loop(0, KPT, chunk_body, 0)


def _stage2(p, q, srcp, dstp):
    mesh = plsc.VectorSubcoreMesh(core_axis_name="c", subcore_axis_name="s", num_cores=NC, num_subcores=NS)
    fn = functools.partial(
        pl.kernel,
        out_type=jax.ShapeDtypeStruct((E_PAD, H), jnp.float32),
        mesh=mesh,
        scratch_types=[
            pltpu.VMEM((KPT, CHUNK), jnp.int32),
            pltpu.VMEM((KPT, CHUNK), jnp.int32),
            pltpu.VMEM((CHUNK, H), jnp.float32),
            pltpu.VMEM((CHUNK, H), jnp.float32),
            pltpu.SemaphoreType.DMA,
            pltpu.SemaphoreType.DMA,
        ],
    )(_gather_body)
    return fn(p, q, srcp, dstp)


def _edge_mlp_body(pre_ref, e2g_ref, fdlf_ref, src_ref, lp_ref, w8_ref,
                   we2_ref, be2_ref, out_ref, cnt_ref):
    oh = (e2g_ref[...] == lax.broadcasted_iota(jnp.int32, (1, G), 1)
          ).astype(jnp.float32)
    sv = src_ref[...]
    oh_hi = (lax.shift_right_logical(sv, 7)
             == lax.broadcasted_iota(jnp.int32, (1, NPB), 1)
             ).astype(jnp.float32)
    oh_lo = (lax.bitwise_and(sv, 127)
             == lax.broadcasted_iota(jnp.int32, (1, CHUNK), 1)
             ).astype(jnp.float32)
    blk_cnt = lax.dot_general(oh_hi, oh_lo, (((0,), (0,)), ((), ())),
                              preferred_element_type=jnp.float32)

    @pl.when(pl.program_id(0) == 0)
    def _init_cnt():
        cnt_ref[...] = jnp.zeros_like(cnt_ref)

    cnt_ref[...] += blk_cnt
    x = (pre_ref[...]
         + jnp.dot(oh, lp_ref[...], preferred_element_type=jnp.float32)
         + jnp.dot(fdlf_ref[...], w8_ref[...],
                   preferred_element_type=jnp.float32))
    h = x * jax.nn.sigmoid(x)
    y = jnp.dot(h, we2_ref[...], preferred_element_type=jnp.float32) \
        + be2_ref[...]
    out_ref[...] = y * jax.nn.sigmoid(y)


_BE = 1024


def _stage3(pre, e2gp, fdlf, srcc, lp, w8, we2, be2r):
    return pl.pallas_call(
        _edge_mlp_body,
        grid=(E_PAD // _BE,),
        in_specs=[
            pl.BlockSpec((_BE, H), lambda i: (i, 0)),
            pl.BlockSpec((_BE, 1), lambda i: (i, 0)),
            pl.BlockSpec((_BE, 8), lambda i: (i, 0)),
            pl.BlockSpec((_BE, 1), lambda i: (i, 0)),
            pl.BlockSpec((G, H), lambda i: (0, 0)),
            pl.BlockSpec((8, H), lambda i: (0, 0)),
            pl.BlockSpec((H, H), lambda i: (0, 0)),
            pl.BlockSpec((1, H), lambda i: (0, 0)),
        ],
        out_specs=[
            pl.BlockSpec((_BE, H), lambda i: (i, 0)),
            pl.BlockSpec((NPB, CHUNK), lambda i: (0, 0)),
        ],
        out_shape=[
            jax.ShapeDtypeStruct((E_PAD, H), jnp.float32),
            jax.ShapeDtypeStruct((NPB, CHUNK), jnp.float32),
        ],
    )(pre, e2gp, fdlf, srcc, lp, w8, we2, be2r)


NR = NP // NC
RA = NR + CHUNK
SPT = RA // NS
CIMG = RA // CHUNK
CIMGP = 48
KPT2 = E_PAD // CHUNK // NS


def _scatter_body(ef_hbm, src_hbm, sums_hbm,
                  idx, idx2, buf, zsum, sum_sh):
    c = lax.axis_index("c")
    s = lax.axis_index("s")
    base = c * NR

    zero16 = jnp.zeros((LANES,), jnp.float32)

    def zrow(r, carry):
        for cc in range(H // LANES):
            zsum[r, pl.ds(cc * LANES, LANES)] = zero16
        return carry

    lax.fori_loop(0, CHUNK, zrow, 0)

    pltpu.sync_copy(src_hbm.at[pl.ds(s * KPT2, KPT2)], idx)

    pltpu.sync_copy(zsum, sum_sh.at[pl.ds(s * SPT, CHUNK)])
    pltpu.sync_copy(zsum, sum_sh.at[pl.ds(s * SPT + CHUNK, CHUNK)])
    pltpu.sync_copy(zsum.at[pl.ds(0, SPT - 2 * CHUNK)],
                    sum_sh.at[pl.ds(s * SPT + 2 * CHUNK, SPT - 2 * CHUNK)])

    plsc.subcore_barrier()

    def chunk_body(j, carry):
        pltpu.sync_copy(ef_hbm.at[pl.ds((s * KPT2 + j) * CHUNK, CHUNK)], buf)
        for g in range(CHUNK // LANES):
            iv = idx[j, pl.ds(g * LANES, LANES)]
            local = (iv - base).astype(jnp.uint32)
            sel = jnp.minimum(local, jnp.uint32(NR)).astype(jnp.int32)
            idx2[pl.ds(g * LANES, LANES)] = sel
        pltpu.sync_copy(buf, sum_sh.at[idx2], add=True)
        return carry

    lax.fori_loop(0, KPT2, chunk_body, 0)
    plsc.subcore_barrier()

    for b, w in ((0, CHUNK), (CHUNK, CHUNK), (2 * CHUNK, SPT - 2 * CHUNK)):
        pltpu.sync_copy(sum_sh.at[pl.ds(s * SPT + b, w)],
                        buf.at[pl.ds(0, w)])
        pltpu.sync_copy(buf.at[pl.ds(0, w)],
                        sums_hbm.at[c, pl.ds(s * SPT + b, w)])


def _stage4(ef, srcp):
    mesh = plsc.VectorSubcoreMesh(core_axis_name="c", subcore_axis_name="s",
                                  num_cores=NC, num_subcores=NS)
    fn = functools.partial(
        pl.kernel,
        out_type=jax.ShapeDtypeStruct((NC, RA, H), jnp.float32),
        mesh=mesh,
        scratch_types=[
            pltpu.VMEM((KPT2, CHUNK), jnp.int32),
            pltpu.VMEM((CHUNK,), jnp.int32),
            pltpu.VMEM((CHUNK, H), jnp.float32),
            pltpu.VMEM((CHUNK, H), jnp.float32),
            pltpu.VMEM_SHARED((RA, H), jnp.float32),
        ],
    )(_scatter_body)
    return fn(ef, srcp)


def _node_mlp_body(nf_ref, s_ref, c_ref,
                   wn1a_ref, wn1b_ref, bn1_ref, wn2_ref, bn2_ref,
                   out_ref):
    nf = nf_ref[...]
    cnt = c_ref[...]
    rec = 1.0 / jnp.maximum(cnt, 1.0)
    agg = s_ref[...] * rec
    g = (jnp.dot(nf, wn1a_ref[...], preferred_element_type=jnp.float32)
         + jnp.dot(agg, wn1b_ref[...], preferred_element_type=jnp.float32)
         + bn1_ref[...])
    g = g * jax.nn.sigmoid(g)
    y = jnp.dot(g, wn2_ref[...], preferred_element_type=jnp.float32) \
        + bn2_ref[...]
    out_ref[...] = nf + y * jax.nn.sigmoid(y)


_BN5 = 2000


def _stage5(nf, sums, cnt, wn1a, wn1b, bn1r, wn2, bn2r):
    return pl.pallas_call(
        _node_mlp_body,
        grid=(N // _BN5,),
        in_specs=[
            pl.BlockSpec((_BN5, H), lambda i: (i, 0)),
            pl.BlockSpec((_BN5, H), lambda i: (i, 0)),
            pl.BlockSpec((_BN5, 1), lambda i: (i, 0)),
            pl.BlockSpec((H, H), lambda i: (0, 0)),
            pl.BlockSpec((H, H), lambda i: (0, 0)),
            pl.BlockSpec((1, H), lambda i: (0, 0)),
            pl.BlockSpec((H, H), lambda i: (0, 0)),
            pl.BlockSpec((1, H), lambda i: (0, 0)),
        ],
        out_specs=pl.BlockSpec((_BN5, H), lambda i: (i, 0)),
        out_shape=jax.ShapeDtypeStruct((N, H), jnp.float32),
    )(nf, sums, cnt, wn1a, wn1b, bn1r, wn2, bn2r)


def kernel(node_features, frac_coords, lattices, edge_index, edge2graph,
           frac_diff, l_f_features, We1, be1, We2, be2, Wn1, bn1, Wn2, bn2):
    f32 = jnp.float32
    nf_pad = jnp.pad(node_features, ((0, NP - N), (0, 0)))
    w1a = We1[0:H]
    w1b = We1[H:2 * H]
    w1c = We1[2 * H:2 * H + 6]
    w8 = jnp.concatenate([We1[2 * H + 6:], jnp.zeros((2, H), f32)], axis=0)
    be1r = be1.reshape(1, H)
    be2r = be2.reshape(1, H)
    bn1r = bn1.reshape(1, H)
    bn2r = bn2.reshape(1, H)
    wn1a = Wn1[0:H]
    wn1b = Wn1[H:2 * H]

    pad_e = E_PAD - E
    src = edge_index[0]
    dst = edge_index[1]
    srcp = jnp.concatenate(
        [src, jnp.full((pad_e,), N, jnp.int32)]).reshape(E_PAD // CHUNK, CHUNK)
    dstp = jnp.concatenate(
        [dst, jnp.full((pad_e,), N, jnp.int32)]).reshape(E_PAD // CHUNK, CHUNK)
    e2gp = jnp.concatenate(
        [edge2graph, jnp.zeros((pad_e,), jnp.int32)]).reshape(E_PAD, 1)
    fdlf = jnp.pad(jnp.concatenate([frac_diff, l_f_features], axis=1),
                   ((0, pad_e), (0, 2)))

    p, q, lp = _stage1(nf_pad, w1a, w1b, lattices, w1c, be1r)
    pre = _stage2(p, q, srcp, dstp)
    srcc = srcp.reshape(E_PAD, 1)
    ef, cnt_img = _stage3(pre, e2gp, fdlf, srcc, lp, w8, We2, be2r)
    sums = _stage4(ef, srcp)
    sums_cat = jnp.concatenate([sums[0, 0:NR], sums[1, 0:NR]], axis=0)
    cnt_col = cnt_img.reshape(NP, 1)
    return _stage5(node_features, sums_cat, cnt_col[0:N],
                   wn1a, wn1b, bn1r, Wn2, bn2r)


# speedup vs baseline: 2.7720x; 1.0648x over previous
import functools

import jax
import jax.numpy as jnp
from jax import lax
from jax.experimental import pallas as pl
from jax.experimental.pallas import tpu as pltpu
from jax.experimental.pallas import tpu_sc as plsc

N = 10000
H = 128
G = 128
E = 320000

NC = 2
NS = 16
NW = NC * NS
CHUNK = 128
KPT = 80
E_PAD = NW * KPT * CHUNK
NP = 10240
NPB = NP // CHUNK
LANES = 16


def _proj_body(nf_ref, w1a_ref, w1b_ref, lat_ref, w1c_ref, be1_ref,
               p_ref, q_ref, lp_ref):
    nf = nf_ref[...]
    p_ref[...] = jnp.dot(nf, w1a_ref[...], preferred_element_type=jnp.float32)
    q_ref[...] = jnp.dot(nf, w1b_ref[...], preferred_element_type=jnp.float32)
    lp_ref[...] = (jnp.dot(lat_ref[...], w1c_ref[...],
                           preferred_element_type=jnp.float32) + be1_ref[...])


_BN1 = 2048


def _stage1(nf_pad, w1a, w1b, lattices, w1c, be1r):
    return pl.pallas_call(
        _proj_body,
        grid=(NP // _BN1,),
        in_specs=[
            pl.BlockSpec((_BN1, H), lambda i: (i, 0)),
            pl.BlockSpec((H, H), lambda i: (0, 0)),
            pl.BlockSpec((H, H), lambda i: (0, 0)),
            pl.BlockSpec((G, 6), lambda i: (0, 0)),
            pl.BlockSpec((6, H), lambda i: (0, 0)),
            pl.BlockSpec((1, H), lambda i: (0, 0)),
        ],
        out_specs=[
            pl.BlockSpec((_BN1, H), lambda i: (i, 0)),
            pl.BlockSpec((_BN1, H), lambda i: (i, 0)),
            pl.BlockSpec((G, H), lambda i: (0, 0)),
        ],
        out_shape=[
            jax.ShapeDtypeStruct((NP, H), jnp.float32),
            jax.ShapeDtypeStruct((NP, H), jnp.float32),
            jax.ShapeDtypeStruct((G, H), jnp.float32),
        ],
    )(nf_pad, w1a, w1b, lattices, w1c, be1r)


def _gather_body(p_hbm, q_hbm, src_hbm, dst_hbm, pre_hbm,
                 sidx, didx, gbufp, gbufq, wbuf,
                 semp0, semq0, semp1, semq1, semw0, semw1):
    c = lax.axis_index("c")
    s = lax.axis_index("s")
    wid = s * NC + c
    base = wid * KPT
    pltpu.sync_copy(src_hbm.at[pl.ds(base, KPT)], sidx)
    pltpu.sync_copy(dst_hbm.at[pl.ds(base, KPT)], didx)

    semp = (semp0, semp1)
    semq = (semq0, semq1)
    semw = (semw0, semw1)

    def gather_desc(j, slot):
        return (pltpu.make_async_copy(p_hbm.at[sidx.at[j]], gbufp.at[slot],
                                      semp[slot]),
                pltpu.make_async_copy(q_hbm.at[didx.at[j]], gbufq.at[slot],
                                      semq[slot]))

    def write_desc(j, slot):
        return pltpu.make_async_copy(
            wbuf.at[slot], pre_hbm.at[pl.ds((base + j) * CHUNK, CHUNK)],
            semw[slot])

    def start_gather(j, slot):
        dp, dq = gather_desc(j, slot)
        dp.start()
        dq.start()

    def step(j, slot, prefetch=True, drain=True):
        dp, dq = gather_desc(j, slot)
        dp.wait()
        dq.wait()
        if prefetch:
            start_gather(j + 1, 1 - slot)
        if drain:
            write_desc(j - 2, slot).wait()

        def row_body(r, carry2):
            for cc in range(H // LANES):
                sl = pl.ds(cc * LANES, LANES)
                wbuf[slot, r, sl] = gbufp[slot, r, sl] + gbufq[slot, r, sl]
            return carry2

        lax.fori_loop(0, CHUNK, row_body, 0)
        write_desc(j, slot).start()

    start_gather(0, 0)
    step(0, 0, drain=False)
    step(1, 1, drain=False)

    def chunk_body(g, carry):
        step(2 * g, 0)
        step(2 * g + 1, 1)
        return carry

    lax.fori_loop(1, KPT // 2 - 1, chunk_body, 0)
    step(KPT - 2, 0)
    step(KPT - 1, 1, prefetch=False)
    write_desc(KPT - 2, 0).wait()
    write_desc(KPT - 1, 1).wait()


def _stage2(p, q, srcp, dstp):
    mesh = plsc.VectorSubcoreMesh(core_axis_name="c", subcore_axis_name="s", num_cores=NC, num_subcores=NS)
    fn = functools.partial(
        pl.kernel,
        out_type=jax.ShapeDtypeStruct((E_PAD, H), jnp.float32),
        mesh=mesh,
        scratch_types=[
            pltpu.VMEM((KPT, CHUNK), jnp.int32),
            pltpu.VMEM((KPT, CHUNK), jnp.int32),
            pltpu.VMEM((2, CHUNK, H), jnp.float32),
            pltpu.VMEM((2, CHUNK, H), jnp.float32),
            pltpu.VMEM((2, CHUNK, H), jnp.float32),
            pltpu.SemaphoreType.DMA,
            pltpu.SemaphoreType.DMA,
            pltpu.SemaphoreType.DMA,
            pltpu.SemaphoreType.DMA,
            pltpu.SemaphoreType.DMA,
            pltpu.SemaphoreType.DMA,
        ],
    )(_gather_body)
    return fn(p, q, srcp, dstp)


def _edge_mlp_body(pre_ref, e2g_ref, fdlf_ref, src_ref, lp_ref,
                   w8_ref, we2_ref, be2_ref, out_ref, cnt_ref):
    oh = (e2g_ref[...] == lax.broadcasted_iota(jnp.int32, (1, G), 1)
          ).astype(jnp.float32)
    sv = src_ref[...]
    oh_hi = (lax.shift_right_logical(sv, 7)
             == lax.broadcasted_iota(jnp.int32, (1, NPB), 1)
             ).astype(jnp.float32)
    oh_lo = (lax.bitwise_and(sv, 127)
             == lax.broadcasted_iota(jnp.int32, (1, CHUNK), 1)
             ).astype(jnp.float32)
    blk_cnt = lax.dot_general(oh_hi, oh_lo, (((0,), (0,)), ((), ())),
                              preferred_element_type=jnp.float32)

    @pl.when(pl.program_id(0) == 0)
    def _init_cnt():
        cnt_ref[...] = jnp.zeros_like(cnt_ref)

    cnt_ref[...] += blk_cnt
    x = (pre_ref[...]
         + jnp.dot(oh, lp_ref[...], preferred_element_type=jnp.float32)
         + jnp.dot(fdlf_ref[...], w8_ref[...],
                   preferred_element_type=jnp.float32))
    h = x * jax.nn.sigmoid(x)
    y = jnp.dot(h, we2_ref[...], preferred_element_type=jnp.float32) \
        + be2_ref[...]
    out_ref[...] = y * jax.nn.sigmoid(y)


_BE = 1024


def _stage3(pre, e2gp, fdlf, srcc, lp, w8, we2, be2r):
    return pl.pallas_call(
        _edge_mlp_body,
        grid=(E_PAD // _BE,),
        in_specs=[
            pl.BlockSpec((_BE, H), lambda i: (i, 0)),
            pl.BlockSpec((_BE, 1), lambda i: (i, 0)),
            pl.BlockSpec((_BE, 8), lambda i: (i, 0)),
            pl.BlockSpec((_BE, 1), lambda i: (i, 0)),
            pl.BlockSpec((G, H), lambda i: (0, 0)),
            pl.BlockSpec((8, H), lambda i: (0, 0)),
            pl.BlockSpec((H, H), lambda i: (0, 0)),
            pl.BlockSpec((1, H), lambda i: (0, 0)),
        ],
        out_specs=[
            pl.BlockSpec((_BE, H), lambda i: (i, 0)),
            pl.BlockSpec((NPB, CHUNK), lambda i: (0, 0)),
        ],
        out_shape=[
            jax.ShapeDtypeStruct((E_PAD, H), jnp.float32),
            jax.ShapeDtypeStruct((NPB, CHUNK), jnp.float32),
        ],
    )(pre, e2gp, fdlf, srcc, lp, w8, we2, be2r)


NR = NP // NC
RA = NR + CHUNK
SPT = RA // NS
KPT2 = E_PAD // CHUNK // NS


def _scatter_body(ef_hbm, src_hbm, sums_hbm,
                  idx, idx2, buf, zsum, sum_sh):
    c = lax.axis_index("c")
    s = lax.axis_index("s")
    base = c * NR

    zero16 = jnp.zeros((LANES,), jnp.float32)

    def zrow(r, carry):
        for cc in range(H // LANES):
            zsum[r, pl.ds(cc * LANES, LANES)] = zero16
        return carry

    lax.fori_loop(0, CHUNK, zrow, 0)

    pltpu.sync_copy(src_hbm.at[pl.ds(s * KPT2, KPT2)], idx)

    pltpu.sync_copy(zsum, sum_sh.at[pl.ds(s * SPT, CHUNK)])
    pltpu.sync_copy(zsum, sum_sh.at[pl.ds(s * SPT + CHUNK, CHUNK)])
    pltpu.sync_copy(zsum.at[pl.ds(0, SPT - 2 * CHUNK)],
                    sum_sh.at[pl.ds(s * SPT + 2 * CHUNK, SPT - 2 * CHUNK)])

    plsc.subcore_barrier()

    def chunk_body(j, carry):
        pltpu.sync_copy(ef_hbm.at[pl.ds((s * KPT2 + j) * CHUNK, CHUNK)], buf)
        for g in range(CHUNK // LANES):
            iv = idx[j, pl.ds(g * LANES, LANES)]
            local = (iv - base).astype(jnp.uint32)
            sel = jnp.minimum(local, jnp.uint32(NR)).astype(jnp.int32)
            idx2[pl.ds(g * LANES, LANES)] = sel
        pltpu.sync_copy(buf, sum_sh.at[idx2], add=True)
        return carry

    lax.fori_loop(0, KPT2, chunk_body, 0)
    plsc.subcore_barrier()

    for b, w in ((0, CHUNK), (CHUNK, CHUNK), (2 * CHUNK, SPT - 2 * CHUNK)):
        pltpu.sync_copy(sum_sh.at[pl.ds(s * SPT + b, w)],
                        buf.at[pl.ds(0, w)])
        pltpu.sync_copy(buf.at[pl.ds(0, w)],
                        sums_hbm.at[c, pl.ds(s * SPT + b, w)])


def _stage4(ef, srcp):
    mesh = plsc.VectorSubcoreMesh(core_axis_name="c", subcore_axis_name="s",
                                  num_cores=NC, num_subcores=NS)
    fn = functools.partial(
        pl.kernel,
        out_type=jax.ShapeDtypeStruct((NC, RA, H), jnp.float32),
        mesh=mesh,
        scratch_types=[
            pltpu.VMEM((KPT2, CHUNK), jnp.int32),
            pltpu.VMEM((CHUNK,), jnp.int32),
            pltpu.VMEM((CHUNK, H), jnp.float32),
            pltpu.VMEM((CHUNK, H), jnp.float32),
            pltpu.VMEM_SHARED((RA, H), jnp.float32),
        ],
    )(_scatter_body)
    return fn(ef, srcp)


def _node_mlp_body(nf_ref, s_ref, c_ref,
                   wn1a_ref, wn1b_ref, bn1_ref, wn2_ref, bn2_ref,
                   out_ref):
    nf = nf_ref[...]
    cnt = c_ref[...]
    rec = 1.0 / jnp.maximum(cnt, 1.0)
    agg = s_ref[...] * rec
    g = (jnp.dot(nf, wn1a_ref[...], preferred_element_type=jnp.float32)
         + jnp.dot(agg, wn1b_ref[...], preferred_element_type=jnp.float32)
         + bn1_ref[...])
    g = g * jax.nn.sigmoid(g)
    y = jnp.dot(g, wn2_ref[...], preferred_element_type=jnp.float32) \
        + bn2_ref[...]
    out_ref[...] = nf + y * jax.nn.sigmoid(y)


_BN5 = 2000


def _stage5(nf, sums, cnt, wn1a, wn1b, bn1r, wn2, bn2r):
    return pl.pallas_call(
        _node_mlp_body,
        grid=(N // _BN5,),
        in_specs=[
            pl.BlockSpec((_BN5, H), lambda i: (i, 0)),
            pl.BlockSpec((_BN5, H), lambda i: (i, 0)),
            pl.BlockSpec((_BN5, 1), lambda i: (i, 0)),
            pl.BlockSpec((H, H), lambda i: (0, 0)),
            pl.BlockSpec((H, H), lambda i: (0, 0)),
            pl.BlockSpec((1, H), lambda i: (0, 0)),
            pl.BlockSpec((H, H), lambda i: (0, 0)),
            pl.BlockSpec((1, H), lambda i: (0, 0)),
        ],
        out_specs=pl.BlockSpec((_BN5, H), lambda i: (i, 0)),
        out_shape=jax.ShapeDtypeStruct((N, H), jnp.float32),
    )(nf, sums, cnt, wn1a, wn1b, bn1r, wn2, bn2r)


def kernel(node_features, frac_coords, lattices, edge_index, edge2graph,
           frac_diff, l_f_features, We1, be1, We2, be2, Wn1, bn1, Wn2, bn2):
    f32 = jnp.float32
    nf_pad = jnp.pad(node_features, ((0, NP - N), (0, 0)))
    w1a = We1[0:H]
    w1b = We1[H:2 * H]
    w1c = We1[2 * H:2 * H + 6]
    w8 = jnp.concatenate([We1[2 * H + 6:], jnp.zeros((2, H), f32)], axis=0)
    be1r = be1.reshape(1, H)
    be2r = be2.reshape(1, H)
    bn1r = bn1.reshape(1, H)
    bn2r = bn2.reshape(1, H)
    wn1a = Wn1[0:H]
    wn1b = Wn1[H:2 * H]

    pad_e = E_PAD - E
    src = edge_index[0]
    dst = edge_index[1]
    srcp = jnp.concatenate(
        [src, jnp.full((pad_e,), N, jnp.int32)]).reshape(E_PAD // CHUNK, CHUNK)
    dstp = jnp.concatenate(
        [dst, jnp.full((pad_e,), N, jnp.int32)]).reshape(E_PAD // CHUNK, CHUNK)
    e2gp = jnp.concatenate(
        [edge2graph, jnp.zeros((pad_e,), jnp.int32)]).reshape(E_PAD, 1)
    fdlf = jnp.pad(jnp.concatenate([frac_diff, l_f_features], axis=1),
                   ((0, pad_e), (0, 2)))

    p, q, lp = _stage1(nf_pad, w1a, w1b, lattices, w1c, be1r)
    pre = _stage2(p, q, srcp, dstp)
    srcc = srcp.reshape(E_PAD, 1)
    ef, cnt_img = _stage3(pre, e2gp, fdlf, srcc, lp, w8, We2, be2r)
    sums = _stage4(ef, srcp)
    sums_cat = jnp.concatenate([sums[0, 0:NR], sums[1, 0:NR]], axis=0)
    cnt_col = cnt_img.reshape(NP, 1)
    return _stage5(node_features, sums_cat, cnt_col[0:N],
                   wn1a, wn1b, bn1r, Wn2, bn2r)
